# baseline (speedup 1.0000x reference)
import jax
import jax.numpy as jnp
from jax.experimental import pallas as pl
from jax.experimental.pallas import tpu as pltpu

_B = 4
_S = 8192
_D = 2048
_N = 64
_STRIDE = 128
_ROWS = _B * _N
_C = 4
_RPC = _ROWS // _C
_HPB = _N // _RPC


def _read(x_hbm, buf, rsem, c):
    b, h = divmod(c, _HPB)
    return pltpu.make_async_copy(
        x_hbm.at[b, pl.ds(h * _RPC, _RPC), 0, :],
        buf.at[pl.ds(c * _RPC, _RPC)],
        rsem.at[c],
    )


def _write(buf, out_hbm, wsem):
    return pltpu.make_async_copy(buf, out_hbm, wsem)


def _gather_body(x_hbm, out_hbm, buf, rsem, wsem):
    for c in range(_C):
        _read(x_hbm, buf, rsem, c).start()
    for c in range(_C):
        _read(x_hbm, buf, rsem, c).wait()
    _write(buf, out_hbm, wsem).start()
    _write(buf, out_hbm, wsem).wait()


def kernel(x):
    x4 = x.reshape(_B, _N, _STRIDE, _D)
    out = pl.pallas_call(
        _gather_body,
        in_specs=[pl.BlockSpec(memory_space=pl.ANY)],
        out_specs=pl.BlockSpec(memory_space=pl.ANY),
        out_shape=jax.ShapeDtypeStruct((_ROWS, _D), jnp.float32),
        scratch_shapes=[
            pltpu.VMEM((_ROWS, _D), jnp.float32),
            pltpu.SemaphoreType.DMA((_C,)),
            pltpu.SemaphoreType.DMA,
        ],
    )(x4)
    return out.reshape(_B, _N, _D)


# speedup vs baseline: 1.1046x; 1.1046x over previous
import jax
import jax.numpy as jnp
from jax.experimental import pallas as pl
from jax.experimental.pallas import tpu as pltpu

_B = 4
_S = 8192
_D = 2048
_N = 64
_STRIDE = 128
_ROWS = _B * _N
_HPB = 2
_RPH = _N // _HPB


def _read(x_hbm, buf, rsem, b, h):
    return pltpu.make_async_copy(
        x_hbm.at[b, pl.ds(h * _RPH, _RPH), 0, :],
        buf.at[pl.ds(b * _N + h * _RPH, _RPH)],
        rsem.at[b * _HPB + h],
    )


def _write(buf, out_hbm, wsem, b):
    return pltpu.make_async_copy(
        buf.at[pl.ds(b * _N, _N)],
        out_hbm.at[pl.ds(b * _N, _N)],
        wsem.at[b],
    )


def _gather_body(x_hbm, out_hbm, buf, rsem, wsem):
    for b in range(_B):
        for h in range(_HPB):
            _read(x_hbm, buf, rsem, b, h).start()
    for b in range(_B):
        for h in range(_HPB):
            _read(x_hbm, buf, rsem, b, h).wait()
        _write(buf, out_hbm, wsem, b).start()
    for b in range(_B):
        _write(buf, out_hbm, wsem, b).wait()


def kernel(x):
    x4 = x.reshape(_B, _N, _STRIDE, _D)
    out = pl.pallas_call(
        _gather_body,
        in_specs=[pl.BlockSpec(memory_space=pl.ANY)],
        out_specs=pl.BlockSpec(memory_space=pl.ANY),
        out_shape=jax.ShapeDtypeStruct((_ROWS, _D), jnp.float32),
        scratch_shapes=[
            pltpu.VMEM((_ROWS, _D), jnp.float32),
            pltpu.SemaphoreType.DMA((_B * _HPB,)),
            pltpu.SemaphoreType.DMA((_B,)),
        ],
    )(x4)
    return out.reshape(_B, _N, _D)


# speedup vs baseline: 1.1275x; 1.0207x over previous
import jax
import jax.numpy as jnp
from jax.experimental import pallas as pl
from jax.experimental.pallas import tpu as pltpu

_B = 4
_S = 8192
_D = 2048
_N = 64
_STRIDE = 128
_ROWS = _B * _N


def _read(x_hbm, buf, rsem, b):
    return pltpu.make_async_copy(
        x_hbm.at[b, :, 0, :],
        buf.at[pl.ds(b * _N, _N)],
        rsem.at[b],
    )


def _write(buf, out_hbm, wsem, b):
    return pltpu.make_async_copy(
        buf.at[pl.ds(b * _N, _N)],
        out_hbm.at[pl.ds(b * _N, _N)],
        wsem.at[b],
    )


def _gather_body(x_hbm, out_hbm, buf, rsem, wsem):
    for b in range(_B):
        _read(x_hbm, buf, rsem, b).start()
    for b in range(_B):
        _read(x_hbm, buf, rsem, b).wait()
        _write(buf, out_hbm, wsem, b).start()
    for b in range(_B):
        _write(buf, out_hbm, wsem, b).wait()


def kernel(x):
    x4 = x.reshape(_B, _N, _STRIDE, _D)
    out = pl.pallas_call(
        _gather_body,
        in_specs=[pl.BlockSpec(memory_space=pl.ANY)],
        out_specs=pl.BlockSpec(memory_space=pl.ANY),
        out_shape=jax.ShapeDtypeStruct((_ROWS, _D), jnp.float32),
        scratch_shapes=[
            pltpu.VMEM((_ROWS, _D), jnp.float32),
            pltpu.SemaphoreType.DMA((_B,)),
            pltpu.SemaphoreType.DMA((_B,)),
        ],
    )(x4)
    return out.reshape(_B, _N, _D)
